# baseline (speedup 1.0000x reference)
import functools

import jax
import jax.numpy as jnp
from jax.experimental import pallas as pl
from jax.experimental.pallas import tpu as pltpu

_BB = 16


def _gs_block(logits_ref, u_ref, out_ref):
    lg = jnp.log(u_ref[...])
    q = lg * lg
    e = jnp.exp(2.0 * logits_ref[...])
    t = e[:, None, :] * (1.0 / q)
    s = jnp.sum(t, axis=2, keepdims=True)
    mn = jnp.min(q * s, axis=1)
    out_ref[...] = e / mn


@jax.jit
def kernel(logits, uniform):
    b, d = logits.shape
    k = uniform.shape[1]
    grid = (b // _BB,)
    return pl.pallas_call(
        _gs_block,
        grid=grid,
        in_specs=[
            pl.BlockSpec((_BB, d), lambda i: (i, 0)),
            pl.BlockSpec((_BB, k, d), lambda i: (i, 0, 0)),
        ],
        out_specs=pl.BlockSpec((_BB, d), lambda i: (i, 0)),
        out_shape=jax.ShapeDtypeStruct((b, d), logits.dtype),
        compiler_params=pltpu.CompilerParams(
            dimension_semantics=("parallel",),
        ),
    )(logits, uniform)


# speedup vs baseline: 1.0646x; 1.0646x over previous
import functools

import jax
import jax.numpy as jnp
from jax.experimental import pallas as pl
from jax.experimental.pallas import tpu as pltpu

_BB = 8


def _gs_block(logits_ref, u_ref, out_ref):
    lg = jnp.log(u_ref[...])
    q = lg * lg
    e = jnp.exp(2.0 * logits_ref[...])
    t = e[:, None, :] * (1.0 / q)
    s = jnp.sum(t, axis=2, keepdims=True)
    mn = jnp.min(q * s, axis=1)
    out_ref[...] = e / mn


@jax.jit
def kernel(logits, uniform):
    b, d = logits.shape
    k = uniform.shape[1]
    grid = (b // _BB,)
    return pl.pallas_call(
        _gs_block,
        grid=grid,
        in_specs=[
            pl.BlockSpec((_BB, d), lambda i: (i, 0)),
            pl.BlockSpec((_BB, k, d), lambda i: (i, 0, 0)),
        ],
        out_specs=pl.BlockSpec((_BB, d), lambda i: (i, 0)),
        out_shape=jax.ShapeDtypeStruct((b, d), logits.dtype),
        compiler_params=pltpu.CompilerParams(
            dimension_semantics=("parallel",),
        ),
    )(logits, uniform)
